# baseline (speedup 1.0000x reference)
import jax
import jax.numpy as jnp
from jax import lax
from jax.experimental import pallas as pl
from jax.experimental.pallas import tpu as pltpu
from jax.experimental.pallas import tpu_sc as plsc

B = 4096
S = 200
D = 64
V = 1_000_000
NC = 2
NS = 16
NW = NC * NS
L = 16

NBLK_FULL = V // 128
BLK_PER_W = NBLK_FULL // NW
NBLK_REM = NBLK_FULL - BLK_PER_W * NW
CHUNKS = S * (B // 128)
CHUNK_PER_W = CHUNKS // NW


def _wid():
    return lax.axis_index("s") * NC + lax.axis_index("c")


def _transpose_body(tok_hbm, scr_hbm, tv0, tv1, g0, g1, rem_v, is0, is1, os0, os1):
    wid = _wid()
    tvs, g1s, isem, osem = (tv0, tv1), (g0, g1), (is0, is1), (os0, os1)
    base = wid * BLK_PER_W

    def stage(col0, b):
        for g in range(8):
            pltpu.async_copy(
                tok_hbm.at[pl.ds(8 * g, 8), pl.ds(col0, 128)],
                tvs[b].at[pl.ds(8 * g, 8)], isem[b])

    def wait_stage(b):
        for g in range(8):
            pltpu.make_async_copy(
                tok_hbm.at[pl.ds(0, 8), pl.ds(0, 128)],
                tvs[b].at[pl.ds(0, 8)], isem[b]).wait()

    def transpose(b, lgroups):
        tv, g1 = tvs[b], g1s[b]
        iot = lax.iota(jnp.int32, L)
        tpat = []
        for l in range(lgroups):
            t16 = l * L + iot
            tpat.append((t16, t16 >> 1, (t16 & 1) * D))

        @plsc.parallel_loop(0, L, 1, unroll=4)
        def _(k):
            dk0 = (iot + k) & (L - 1)
            for d0 in range(0, D, L):
                dk = d0 + dk0
                for t16, rowi, tpar in tpat:
                    vec = plsc.load_gather(tv, [dk, t16])
                    plsc.store_scatter(g1, [rowi, tpar + dk], vec)

    def wr_copy(row0, b, rows=64):
        return pltpu.make_async_copy(
            g1s[b].at[pl.ds(0, rows)], scr_hbm.at[pl.ds(row0, rows)], osem[b])

    def step(blk, b, wr_wait):
        wait_stage(b)
        stage((blk + 1) * 128, 1 - b)
        if wr_wait:
            wr_copy(0, b).wait()
        transpose(b, 8)
        wr_copy(blk * 64, b).start()

    stage(base * 128, 0)
    step(base, 0, False)
    step(base + 1, 1, False)

    def pair(g, carry):
        blk = base + 2 * g
        step(blk, 0, True)
        step(blk + 1, 1, True)
        return carry

    lax.fori_loop(1, BLK_PER_W // 2, pair, 0)
    wait_stage(0)
    wr_copy(0, 0).wait()
    wr_copy(0, 1).wait()

    @pl.when(wid < NBLK_REM)
    def _():
        blk = NBLK_FULL - NBLK_REM + wid
        stage(blk * 128, 0)
        wait_stage(0)
        transpose(0, 8)
        wr_copy(blk * 64, 0).start()
        wr_copy(blk * 64, 0).wait()

    @pl.when(wid == NBLK_REM)
    def _():
        for g in range(8):
            pltpu.async_copy(
                tok_hbm.at[pl.ds(8 * g, 8), pl.ds(NBLK_FULL * 128, D)],
                rem_v.at[pl.ds(8 * g, 8)], is0)
        for g in range(8):
            pltpu.make_async_copy(
                tok_hbm.at[pl.ds(8 * g, 8), pl.ds(NBLK_FULL * 128, D)],
                rem_v.at[pl.ds(8 * g, 8)], is0).wait()
        iot = lax.iota(jnp.int32, L)
        tpat = []
        for l in range(4):
            t16 = l * L + iot
            tpat.append((t16, t16 >> 1, (t16 & 1) * D))
        @plsc.parallel_loop(0, L, 1, unroll=2)
        def _(k):
            dk0 = (iot + k) & (L - 1)
            for d0 in range(0, D, L):
                dk = d0 + dk0
                for t16, rowi, tpar in tpat:
                    vec = plsc.load_gather(rem_v, [dk, t16])
                    plsc.store_scatter(g1s[0], [rowi, tpar + dk], vec)
        wr_copy(NBLK_FULL * 64, 0, 32).start()
        wr_copy(NBLK_FULL * 64, 0, 32).wait()


def _gather_body(ids_hbm, scr_hbm, pos_hbm, out_hbm,
                 pos_v, ix0, ix1, ix2, pr0, pr1, pr2,
                 gb0, gb1, gb2, ob0, ob1, ob2,
                 xs0, xs1, xs2, gs0, gs1, gs2, os0, os1, os2):
    wid = _wid()
    ixs, prs = (ix0, ix1, ix2), (pr0, pr1, pr2)
    gbs, obs = (gb0, gb1, gb2), (ob0, ob1, ob2)
    xsem, gsem, osem = (xs0, xs1, xs2), (gs0, gs1, gs2), (os0, os1, os2)

    pltpu.sync_copy(pos_hbm, pos_v)
    base = wid * CHUNK_PER_W

    def sb(c):
        return c >> 5, (c & 31) * 128

    def ix_copy(c, b):
        s, b0 = sb(c)
        return pltpu.make_async_copy(ids_hbm.at[s, pl.ds(b0, 128)], ixs[b],
                                     xsem[b])

    def gather_copy(b):
        return pltpu.make_async_copy(scr_hbm.at[prs[b]], gbs[b], gsem[b])

    def out_copy(c, b):
        s, b0 = sb(c)
        return pltpu.make_async_copy(
            obs[b], out_hbm.at[s, pl.ds(0, D), pl.ds(b0, 128)], osem[b])

    def make_pairs(b):
        ix, pr = ixs[b], prs[b]
        for i in range(8):
            sl = pl.ds(i * L, L)
            pr[sl] = ix[sl] >> 1

    def body(c, b):
        s, _ = sb(c)
        g, o = gbs[b], obs[b]
        iot = lax.iota(jnp.int32, L)
        s16 = jnp.full((L,), s, jnp.int32)
        rp = []
        for l in range(8):
            r16 = l * L + iot
            par = (ixs[b][pl.ds(l * L, L)] & 1) * D
            rp.append((r16, par))

        @plsc.parallel_loop(0, L, 1, unroll=2)
        def _(k):
            dk0 = (iot + k) & (L - 1)
            for d0 in range(0, D, L):
                dk = d0 + dk0
                ps = plsc.load_gather(pos_v, [s16, dk])
                for r16, par in rp:
                    vec = plsc.load_gather(g, [r16, par + dk])
                    plsc.store_scatter(o, [dk, r16], vec + ps)

    def step(c, b, out_wait, ix_pref, gather_pref):
        gather_copy(b).wait()
        if out_wait:
            out_copy(c - 3, b).wait()
        body(c, b)
        out_copy(c, b).start()
        if ix_pref:
            ix_copy(c + 3, b).start()
        if gather_pref:
            b2 = (b + 2) % 3
            ix_copy(c + 2, b2).wait()
            make_pairs(b2)
            gather_copy(b2).start()

    ix_copy(base, 0).start()
    ix_copy(base + 1, 1).start()
    ix_copy(base + 2, 2).start()
    ix_copy(base, 0).wait()
    make_pairs(0)
    gather_copy(0).start()
    ix_copy(base + 1, 1).wait()
    make_pairs(1)
    gather_copy(1).start()
    step(base, 0, False, True, True)
    step(base + 1, 1, False, True, True)
    step(base + 2, 2, False, True, True)

    def trio(g, carry):
        c = base + 3 * g
        step(c, 0, True, True, True)
        step(c + 1, 1, True, True, True)
        step(c + 2, 2, True, True, True)
        return carry

    lax.fori_loop(1, CHUNK_PER_W // 3 - 1, trio, 0)
    c0 = base + CHUNK_PER_W - 5
    step(c0, 0, True, True, True)
    step(c0 + 1, 1, True, True, True)
    step(c0 + 2, 2, True, False, True)
    step(c0 + 3, 0, True, False, False)
    step(c0 + 4, 1, True, False, False)
    out_copy(c0 + 2, 2).wait()
    out_copy(c0 + 3, 0).wait()
    out_copy(c0 + 4, 1).wait()


def kernel(input_ids, token_table, pos_table):
    idsT = input_ids.T.astype(jnp.int32)
    tokT = token_table.T
    mesh = plsc.VectorSubcoreMesh(core_axis_name="c", subcore_axis_name="s")
    cp = pltpu.CompilerParams(use_tc_tiling_on_sc=True, needs_layout_passes=False,
                              disable_bounds_checks=True)

    f1 = pl.kernel(
        _transpose_body,
        mesh=mesh,
        compiler_params=cp,
        out_type=jax.ShapeDtypeStruct((V // 2, 128), jnp.float32),
        scratch_types=(
            [pltpu.VMEM((D, 128), jnp.float32) for _ in range(4)]
            + [pltpu.VMEM((D, D), jnp.float32)]
            + [pltpu.SemaphoreType.DMA for _ in range(4)]
        ),
    )
    scr = f1(tokT)

    f2 = pl.kernel(
        _gather_body,
        mesh=mesh,
        compiler_params=cp,
        out_type=jax.ShapeDtypeStruct((S, D, B), jnp.float32),
        scratch_types=(
            [pltpu.VMEM((S, D), jnp.float32)]
            + [pltpu.VMEM((128,), jnp.int32) for _ in range(6)]
            + [pltpu.VMEM((128, 128), jnp.float32) for _ in range(3)]
            + [pltpu.VMEM((D, 128), jnp.float32) for _ in range(3)]
            + [pltpu.SemaphoreType.DMA for _ in range(9)]
        ),
    )
    outT = f2(idsT, scr, pos_table)
    return outT.transpose(2, 0, 1)


# speedup vs baseline: 1.0464x; 1.0464x over previous
import jax
import jax.numpy as jnp
from jax import lax
from jax.experimental import pallas as pl
from jax.experimental.pallas import tpu as pltpu
from jax.experimental.pallas import tpu_sc as plsc

B = 4096
S = 200
D = 64
V = 1_000_000
NC = 2
NS = 16
NW = NC * NS
L = 16

NBLK_FULL = V // 128
BLK_PER_W = NBLK_FULL // NW
NBLK_REM = NBLK_FULL - BLK_PER_W * NW
CHUNKS = S * (B // 128)
CHUNK_PER_W = CHUNKS // NW


def _wid():
    return lax.axis_index("s") * NC + lax.axis_index("c")


def _transpose_body(tok_hbm, scr_hbm, tv0, tv1, g0, g1, rem_v, is0, is1, os0, os1):
    wid = _wid()
    tvs, g1s, isem, osem = (tv0, tv1), (g0, g1), (is0, is1), (os0, os1)
    base = wid * BLK_PER_W

    def stage(col0, b):
        for g in range(8):
            pltpu.async_copy(
                tok_hbm.at[pl.ds(8 * g, 8), pl.ds(col0, 128)],
                tvs[b].at[pl.ds(8 * g, 8)], isem[b])

    def wait_stage(b):
        for g in range(8):
            pltpu.make_async_copy(
                tok_hbm.at[pl.ds(0, 8), pl.ds(0, 128)],
                tvs[b].at[pl.ds(0, 8)], isem[b]).wait()

    def transpose(b, lgroups):
        tv, g1 = tvs[b], g1s[b]
        iot = lax.iota(jnp.int32, L)
        tpat = []
        for l in range(lgroups):
            t16 = l * L + iot
            tpat.append((t16, t16 >> 1, (t16 & 1) * D))

        @plsc.parallel_loop(0, L, 1, unroll=2)
        def _(k):
            dk0 = (iot + k) & (L - 1)
            for d0 in range(0, D, L):
                dk = d0 + dk0
                for t16, rowi, tpar in tpat:
                    vec = plsc.load_gather(tv, [dk, t16])
                    plsc.store_scatter(g1, [rowi, tpar + dk], vec)

    def wr_copy(row0, b, rows=64):
        return pltpu.make_async_copy(
            g1s[b].at[pl.ds(0, rows)], scr_hbm.at[pl.ds(row0, rows)], osem[b])

    def step(blk, b, wr_wait):
        wait_stage(b)
        stage((blk + 1) * 128, 1 - b)
        if wr_wait:
            wr_copy(0, b).wait()
        transpose(b, 8)
        wr_copy(blk * 64, b).start()

    stage(base * 128, 0)
    step(base, 0, False)
    step(base + 1, 1, False)

    def pair(g, carry):
        blk = base + 2 * g
        step(blk, 0, True)
        step(blk + 1, 1, True)
        return carry

    lax.fori_loop(1, BLK_PER_W // 2, pair, 0)
    wait_stage(0)
    wr_copy(0, 0).wait()
    wr_copy(0, 1).wait()

    @pl.when(wid < NBLK_REM)
    def _():
        blk = NBLK_FULL - NBLK_REM + wid
        stage(blk * 128, 0)
        wait_stage(0)
        transpose(0, 8)
        wr_copy(blk * 64, 0).start()
        wr_copy(blk * 64, 0).wait()

    @pl.when(wid == NBLK_REM)
    def _():
        for g in range(8):
            pltpu.async_copy(
                tok_hbm.at[pl.ds(8 * g, 8), pl.ds(NBLK_FULL * 128, D)],
                rem_v.at[pl.ds(8 * g, 8)], is0)
        for g in range(8):
            pltpu.make_async_copy(
                tok_hbm.at[pl.ds(8 * g, 8), pl.ds(NBLK_FULL * 128, D)],
                rem_v.at[pl.ds(8 * g, 8)], is0).wait()
        iot = lax.iota(jnp.int32, L)
        tpat = []
        for l in range(4):
            t16 = l * L + iot
            tpat.append((t16, t16 >> 1, (t16 & 1) * D))
        @plsc.parallel_loop(0, L, 1, unroll=2)
        def _(k):
            dk0 = (iot + k) & (L - 1)
            for d0 in range(0, D, L):
                dk = d0 + dk0
                for t16, rowi, tpar in tpat:
                    vec = plsc.load_gather(rem_v, [dk, t16])
                    plsc.store_scatter(g1s[0], [rowi, tpar + dk], vec)
        wr_copy(NBLK_FULL * 64, 0, 32).start()
        wr_copy(NBLK_FULL * 64, 0, 32).wait()


def _gather_body(ids_hbm, scr_hbm, pos_hbm, out_hbm,
                 pos_v, ix0, ix1, ix2, pr0, pr1, pr2,
                 gb0, gb1, gb2, ob0, ob1, ob2,
                 xs0, xs1, xs2, gs0, gs1, gs2, os0, os1, os2):
    wid = _wid()
    ixs, prs = (ix0, ix1, ix2), (pr0, pr1, pr2)
    gbs, obs = (gb0, gb1, gb2), (ob0, ob1, ob2)
    xsem, gsem, osem = (xs0, xs1, xs2), (gs0, gs1, gs2), (os0, os1, os2)

    pltpu.sync_copy(pos_hbm, pos_v)
    base = wid * CHUNK_PER_W

    def sb(c):
        return c >> 5, (c & 31) * 128

    def ix_copy(c, b):
        s, b0 = sb(c)
        return pltpu.make_async_copy(ids_hbm.at[s, pl.ds(b0, 128)], ixs[b],
                                     xsem[b])

    def gather_copy(b):
        return pltpu.make_async_copy(scr_hbm.at[prs[b]], gbs[b], gsem[b])

    def out_copy(c, b):
        s, b0 = sb(c)
        return pltpu.make_async_copy(
            obs[b], out_hbm.at[s, pl.ds(0, D), pl.ds(b0, 128)], osem[b])

    def make_pairs(b):
        ix, pr = ixs[b], prs[b]
        for i in range(8):
            sl = pl.ds(i * L, L)
            pr[sl] = ix[sl] >> 1

    def body(c, b):
        s, _ = sb(c)
        g, o = gbs[b], obs[b]
        iot = lax.iota(jnp.int32, L)
        s16 = jnp.full((L,), s, jnp.int32)
        rp = []
        for l in range(8):
            r16 = l * L + iot
            par = (ixs[b][pl.ds(l * L, L)] & 1) * D
            rp.append((r16, par))

        @plsc.parallel_loop(0, L, 1, unroll=2)
        def _(k):
            dk0 = (iot + k) & (L - 1)
            for d0 in range(0, D, L):
                dk = d0 + dk0
                ps = plsc.load_gather(pos_v, [s16, dk])
                for r16, par in rp:
                    vec = plsc.load_gather(g, [r16, par + dk])
                    plsc.store_scatter(o, [dk, r16], vec + ps)

    def step(c, b, out_wait, ix_pref, gather_pref):
        gather_copy(b).wait()
        if out_wait:
            out_copy(c - 3, b).wait()
        body(c, b)
        out_copy(c, b).start()
        if ix_pref:
            ix_copy(c + 3, b).start()
        if gather_pref:
            b2 = (b + 2) % 3
            ix_copy(c + 2, b2).wait()
            make_pairs(b2)
            gather_copy(b2).start()

    ix_copy(base, 0).start()
    ix_copy(base + 1, 1).start()
    ix_copy(base + 2, 2).start()
    ix_copy(base, 0).wait()
    make_pairs(0)
    gather_copy(0).start()
    ix_copy(base + 1, 1).wait()
    make_pairs(1)
    gather_copy(1).start()
    step(base, 0, False, True, True)
    step(base + 1, 1, False, True, True)
    step(base + 2, 2, False, True, True)

    def trio(g, carry):
        c = base + 3 * g
        step(c, 0, True, True, True)
        step(c + 1, 1, True, True, True)
        step(c + 2, 2, True, True, True)
        return carry

    lax.fori_loop(1, CHUNK_PER_W // 3 - 1, trio, 0)
    c0 = base + CHUNK_PER_W - 5
    step(c0, 0, True, True, True)
    step(c0 + 1, 1, True, True, True)
    step(c0 + 2, 2, True, False, True)
    step(c0 + 3, 0, True, False, False)
    step(c0 + 4, 1, True, False, False)
    out_copy(c0 + 2, 2).wait()
    out_copy(c0 + 3, 0).wait()
    out_copy(c0 + 4, 1).wait()


def kernel(input_ids, token_table, pos_table):
    idsT = input_ids.T.astype(jnp.int32)
    tokT = token_table.T
    mesh = plsc.VectorSubcoreMesh(core_axis_name="c", subcore_axis_name="s")
    cp = pltpu.CompilerParams(use_tc_tiling_on_sc=True, needs_layout_passes=False,
                              disable_bounds_checks=True)

    f1 = pl.kernel(
        _transpose_body,
        mesh=mesh,
        compiler_params=cp,
        out_type=jax.ShapeDtypeStruct((V // 2, 128), jnp.float32),
        scratch_types=(
            [pltpu.VMEM((D, 128), jnp.float32) for _ in range(4)]
            + [pltpu.VMEM((D, D), jnp.float32)]
            + [pltpu.SemaphoreType.DMA for _ in range(4)]
        ),
    )
    scr = f1(tokT)

    f2 = pl.kernel(
        _gather_body,
        mesh=mesh,
        compiler_params=cp,
        out_type=jax.ShapeDtypeStruct((S, D, B), jnp.float32),
        scratch_types=(
            [pltpu.VMEM((S, D), jnp.float32)]
            + [pltpu.VMEM((128,), jnp.int32) for _ in range(6)]
            + [pltpu.VMEM((128, 128), jnp.float32) for _ in range(3)]
            + [pltpu.VMEM((D, 128), jnp.float32) for _ in range(3)]
            + [pltpu.SemaphoreType.DMA for _ in range(9)]
        ),
    )
    outT = f2(idsT, scr, pos_table)
    return outT.transpose(2, 0, 1)
